# Initial kernel scaffold; baseline (speedup 1.0000x reference)
import jax
import jax.numpy as jnp
from jax.experimental import pallas as pl


def kernel(index, features):
    raise NotImplementedError("write your pallas kernel here")


# speedup vs baseline: 2.7592x; 2.7592x over previous
import functools

import jax
import jax.numpy as jnp
from jax import lax
from jax.experimental import pallas as pl
from jax.experimental.pallas import tpu as pltpu
from jax.experimental.pallas import tpu_sc as plsc

_NC, _NS = 2, 16
_NW = _NC * _NS
_D = 128
_CHUNK = 128


@functools.lru_cache(maxsize=None)
def _make_gather(B: int):
    assert B % (_NW * _CHUNK) == 0
    b_per_w = B // _NW
    n_chunks = b_per_w // _CHUNK
    mesh = plsc.VectorSubcoreMesh(core_axis_name="c", subcore_axis_name="s")

    @functools.partial(
        pl.kernel,
        out_type=jax.ShapeDtypeStruct((B, _D), jnp.float32),
        mesh=mesh,
        scratch_types=[
            pltpu.VMEM((_CHUNK,), jnp.int32),
            pltpu.VMEM((_CHUNK, _D), jnp.float32),
            pltpu.SemaphoreType.DMA,
        ],
    )
    def gather_kernel(idx_hbm, table_hbm, out_hbm, idx_v, rows_v, sem):
        wid = lax.axis_index("s") * _NC + lax.axis_index("c")
        base = wid * b_per_w

        def body(j, carry):
            off = base + j * _CHUNK
            pltpu.sync_copy(idx_hbm.at[pl.ds(off, _CHUNK)], idx_v)
            pltpu.async_copy(table_hbm.at[idx_v], rows_v, sem).wait()
            pltpu.sync_copy(rows_v, out_hbm.at[pl.ds(off, _CHUNK)])
            return carry

        lax.fori_loop(0, n_chunks, body, 0)

    return gather_kernel


def kernel(index, features):
    n, s = index.shape
    out = _make_gather(n * s)(index.reshape(-1), features)
    return out.reshape(n, s, _D)


# speedup vs baseline: 3.3370x; 1.2094x over previous
import functools

import jax
import jax.numpy as jnp
from jax import lax
from jax.experimental import pallas as pl
from jax.experimental.pallas import tpu as pltpu
from jax.experimental.pallas import tpu_sc as plsc

_NC, _NS = 2, 16
_NW = _NC * _NS
_D = 128
_CHUNK = 128
_NBUF = 5


@functools.lru_cache(maxsize=None)
def _make_gather(B: int):
    assert B % (_NW * _CHUNK * _NBUF) == 0
    b_per_w = B // _NW
    n_chunks = b_per_w // _CHUNK
    n_groups = n_chunks // _NBUF
    mesh = plsc.VectorSubcoreMesh(core_axis_name="c", subcore_axis_name="s")

    @functools.partial(
        pl.kernel,
        out_type=jax.ShapeDtypeStruct((B, _D), jnp.float32),
        mesh=mesh,
        scratch_types=[
            pltpu.VMEM((_NBUF, _CHUNK), jnp.int32),
            pltpu.VMEM((_NBUF, _CHUNK, _D), jnp.float32),
            pltpu.SemaphoreType.DMA((_NBUF,)),
            pltpu.SemaphoreType.DMA((_NBUF,)),
            pltpu.SemaphoreType.DMA((_NBUF,)),
        ],
    )
    def gather_kernel(idx_hbm, table_hbm, out_hbm, idx_v, rows_v,
                      sem_i, sem_g, sem_s):
        wid = lax.axis_index("s") * _NC + lax.axis_index("c")
        base = wid * b_per_w

        def idx_copy(g, b):
            off = base + (g * _NBUF + b) * _CHUNK
            return pltpu.make_async_copy(
                idx_hbm.at[pl.ds(off, _CHUNK)], idx_v.at[b], sem_i.at[b])

        def gather(b):
            return pltpu.make_async_copy(
                table_hbm.at[idx_v.at[b]], rows_v.at[b], sem_g.at[b])

        def store(g, b):
            off = base + (g * _NBUF + b) * _CHUNK
            return pltpu.make_async_copy(
                rows_v.at[b], out_hbm.at[pl.ds(off, _CHUNK)], sem_s.at[b])

        for b in range(_NBUF):
            idx_copy(0, b).start()

        def group(g, carry):
            for b in range(_NBUF):
                idx_copy(g, b).wait()

                @pl.when(g > 0)
                def _():
                    store(g - 1, b).wait()

                gather(b).start()
            for b in range(_NBUF):
                gather(b).wait()
                store(g, b).start()

                @pl.when(g + 1 < n_groups)
                def _():
                    idx_copy(g + 1, b).start()
            return carry

        lax.fori_loop(0, n_groups, group, 0)
        for b in range(_NBUF):
            store(n_groups - 1, b).wait()

    return gather_kernel


def kernel(index, features):
    n, s = index.shape
    out = _make_gather(n * s)(index.reshape(-1), features)
    return out.reshape(n, s, _D)


# speedup vs baseline: 5.9064x; 1.7699x over previous
import functools

import jax
import jax.numpy as jnp
from jax import lax
from jax.experimental import pallas as pl
from jax.experimental.pallas import tpu as pltpu
from jax.experimental.pallas import tpu_sc as plsc

_NC, _NS = 2, 16
_NW = _NC * _NS
_D = 128
_CB = 4
_NBUF = 4


@functools.lru_cache(maxsize=None)
def _make_gather(N: int, S: int):
    assert N % (_NW * _CB * _NBUF) == 0
    b_per_w = N // _NW
    n_chunks = b_per_w // _CB
    n_groups = n_chunks // _NBUF
    mesh = plsc.VectorSubcoreMesh(core_axis_name="c", subcore_axis_name="s")

    @functools.partial(
        pl.kernel,
        out_type=jax.ShapeDtypeStruct((N, S, _D), jnp.float32),
        mesh=mesh,
        scratch_types=[
            pltpu.VMEM((_NBUF, _CB, S), jnp.int32),
            pltpu.VMEM((_NBUF, _CB, S, _D), jnp.float32),
            pltpu.SemaphoreType.DMA((_NBUF,)),
            pltpu.SemaphoreType.DMA((_NBUF,)),
            pltpu.SemaphoreType.DMA((_NBUF,)),
        ],
    )
    def gather_kernel(idx_hbm, table_hbm, out_hbm, idx_v, rows_v,
                      sem_i, sem_g, sem_s):
        wid = lax.axis_index("s") * _NC + lax.axis_index("c")
        base = wid * b_per_w

        def idx_copy(g, b):
            off = base + (g * _NBUF + b) * _CB
            return pltpu.make_async_copy(
                idx_hbm.at[pl.ds(off, _CB)], idx_v.at[b], sem_i.at[b])

        def gather(b, j):
            return pltpu.make_async_copy(
                table_hbm.at[idx_v.at[b, j]], rows_v.at[b, j], sem_g.at[b])

        def store(g, b):
            off = base + (g * _NBUF + b) * _CB
            return pltpu.make_async_copy(
                rows_v.at[b], out_hbm.at[pl.ds(off, _CB)], sem_s.at[b])

        for b in range(_NBUF):
            idx_copy(0, b).start()

        def group(g, carry):
            for b in range(_NBUF):
                idx_copy(g, b).wait()

                @pl.when(g > 0)
                def _():
                    store(g - 1, b).wait()

                for j in range(_CB):
                    gather(b, j).start()
            for b in range(_NBUF):
                for j in range(_CB):
                    gather(b, j).wait()
                store(g, b).start()

                @pl.when(g + 1 < n_groups)
                def _():
                    idx_copy(g + 1, b).start()
            return carry

        lax.fori_loop(0, n_groups, group, 0)
        for b in range(_NBUF):
            store(n_groups - 1, b).wait()

    return gather_kernel


def kernel(index, features):
    n, s = index.shape
    return _make_gather(n, s)(index, features)


# speedup vs baseline: 5.9130x; 1.0011x over previous
import functools

import jax
import jax.numpy as jnp
from jax import lax
from jax.experimental import pallas as pl
from jax.experimental.pallas import tpu as pltpu
from jax.experimental.pallas import tpu_sc as plsc

_NC, _NS = 2, 16
_NW = _NC * _NS
_D = 128
_CB = 4
_NBUF = 4


@functools.lru_cache(maxsize=None)
def _make_gather(N: int, S: int):
    assert N % (_NW * _CB * _NBUF) == 0
    b_per_w = N // _NW
    n_chunks = b_per_w // _CB
    n_groups = n_chunks // _NBUF
    mesh = plsc.VectorSubcoreMesh(core_axis_name="c", subcore_axis_name="s")

    @functools.partial(
        pl.kernel,
        out_type=jax.ShapeDtypeStruct((N, S, _D), jnp.float32),
        mesh=mesh,
        compiler_params=pltpu.CompilerParams(use_tc_tiling_on_sc=True),
        scratch_types=[
            pltpu.VMEM((_NBUF, _CB, S), jnp.int32),
            pltpu.VMEM((_NBUF, _CB, S, _D), jnp.float32),
            pltpu.SemaphoreType.DMA((_NBUF,)),
            pltpu.SemaphoreType.DMA((_NBUF,)),
            pltpu.SemaphoreType.DMA((_NBUF,)),
        ],
    )
    def gather_kernel(idx_hbm, table_hbm, out_hbm, idx_v, rows_v,
                      sem_i, sem_g, sem_s):
        wid = lax.axis_index("s") * _NC + lax.axis_index("c")
        base = wid * b_per_w

        def idx_copy(g, b):
            off = base + (g * _NBUF + b) * _CB
            return pltpu.make_async_copy(
                idx_hbm.at[pl.ds(off, _CB)], idx_v.at[b], sem_i.at[b])

        def gather(b, j):
            return pltpu.make_async_copy(
                table_hbm.at[idx_v.at[b, j]], rows_v.at[b, j], sem_g.at[b])

        def store(g, b):
            off = base + (g * _NBUF + b) * _CB
            return pltpu.make_async_copy(
                rows_v.at[b], out_hbm.at[pl.ds(off, _CB)], sem_s.at[b])

        for b in range(_NBUF):
            idx_copy(0, b).start()

        def group(g, carry):
            for b in range(_NBUF):
                idx_copy(g, b).wait()

                @pl.when(g > 0)
                def _():
                    store(g - 1, b).wait()

                for j in range(_CB):
                    gather(b, j).start()
            for b in range(_NBUF):
                for j in range(_CB):
                    gather(b, j).wait()
                store(g, b).start()

                @pl.when(g + 1 < n_groups)
                def _():
                    idx_copy(g + 1, b).start()
            return carry

        lax.fori_loop(0, n_groups, group, 0)
        for b in range(_NBUF):
            store(n_groups - 1, b).wait()

    return gather_kernel


def kernel(index, features):
    n, s = index.shape
    return _make_gather(n, s)(index, features)


# speedup vs baseline: 10.4136x; 1.7612x over previous
import functools

import jax
import jax.numpy as jnp
from jax import lax
from jax.experimental import pallas as pl
from jax.experimental.pallas import tpu as pltpu
from jax.experimental.pallas import tpu_sc as plsc

_NC, _NS = 2, 16
_NW = _NC * _NS
_D = 128
_BW = 128
_NBUF = 5


@functools.lru_cache(maxsize=None)
def _make_gather(S: int, N: int):
    assert N == _NW * _BW and S % _NBUF == 0
    n_groups = S // _NBUF
    mesh = plsc.VectorSubcoreMesh(core_axis_name="c", subcore_axis_name="s")

    @functools.partial(
        pl.kernel,
        out_type=jax.ShapeDtypeStruct((S, N, _D), jnp.float32),
        mesh=mesh,
        scratch_types=[
            pltpu.VMEM((S, _BW), jnp.int32),
            pltpu.VMEM((_NBUF, _BW, _D), jnp.float32),
            pltpu.SemaphoreType.DMA((_NBUF,)),
            pltpu.SemaphoreType.DMA((_NBUF,)),
        ],
    )
    def gather_kernel(idx_hbm, table_hbm, out_hbm, idx_v, rows_v,
                      sem_g, sem_s):
        wid = lax.axis_index("s") * _NC + lax.axis_index("c")
        b0 = wid * _BW

        pltpu.sync_copy(idx_hbm.at[pl.ds(0, S), pl.ds(b0, _BW)], idx_v)

        def gather(g, b):
            return pltpu.make_async_copy(
                table_hbm.at[idx_v.at[g * _NBUF + b]], rows_v.at[b],
                sem_g.at[b])

        def store(g, b):
            return pltpu.make_async_copy(
                rows_v.at[b], out_hbm.at[g * _NBUF + b, pl.ds(b0, _BW)],
                sem_s.at[b])

        def group(g, carry):
            for b in range(_NBUF):
                @pl.when(g > 0)
                def _():
                    store(g - 1, b).wait()

                gather(g, b).start()
            for b in range(_NBUF):
                gather(g, b).wait()
                store(g, b).start()
            return carry

        lax.fori_loop(0, n_groups, group, 0)
        for b in range(_NBUF):
            store(n_groups - 1, b).wait()

    return gather_kernel


def kernel(index, features):
    n, s = index.shape
    out_t = _make_gather(s, n)(index.T, features)
    return jnp.transpose(out_t, (1, 0, 2))


# speedup vs baseline: 10.7077x; 1.0282x over previous
import functools

import jax
import jax.numpy as jnp
from jax import lax
from jax.experimental import pallas as pl
from jax.experimental.pallas import tpu as pltpu
from jax.experimental.pallas import tpu_sc as plsc

_NC, _NS = 2, 16
_NW = _NC * _NS
_D = 128
_BW = 128
_CW = 64
_NH = _BW // _CW
_NBUF = 10


@functools.lru_cache(maxsize=None)
def _make_gather(S: int, N: int):
    assert N == _NW * _BW and (S * _NH) % _NBUF == 0
    n_chunks = S * _NH
    n_groups = n_chunks // _NBUF
    mesh = plsc.VectorSubcoreMesh(core_axis_name="c", subcore_axis_name="s")

    @functools.partial(
        pl.kernel,
        out_type=jax.ShapeDtypeStruct((S, N, _D), jnp.float32),
        mesh=mesh,
        scratch_types=[
            pltpu.VMEM((_NBUF, _CW), jnp.int32),
            pltpu.VMEM((_NBUF, _CW, _D), jnp.float32),
            pltpu.SemaphoreType.DMA((_NBUF,)),
            pltpu.SemaphoreType.DMA((_NBUF,)),
            pltpu.SemaphoreType.DMA((_NBUF,)),
        ],
    )
    def gather_kernel(idx_hbm, table_hbm, out_hbm, idx_v, rows_v,
                      sem_i, sem_g, sem_s):
        wid = lax.axis_index("s") * _NC + lax.axis_index("c")
        b0 = wid * _BW

        def chunk_pos(g, b):
            k = g * _NBUF + b
            return k // _NH, b0 + (k % _NH) * _CW

        def idx_copy(g, b):
            s, co = chunk_pos(g, b)
            return pltpu.make_async_copy(
                idx_hbm.at[s, pl.ds(co, _CW)], idx_v.at[b], sem_i.at[b])

        def gather(b):
            return pltpu.make_async_copy(
                table_hbm.at[idx_v.at[b]], rows_v.at[b], sem_g.at[b])

        def store(g, b):
            s, co = chunk_pos(g, b)
            return pltpu.make_async_copy(
                rows_v.at[b], out_hbm.at[s, pl.ds(co, _CW)], sem_s.at[b])

        for b in range(_NBUF):
            idx_copy(0, b).start()

        def group(g, carry):
            for b in range(_NBUF):
                idx_copy(g, b).wait()

                @pl.when(g > 0)
                def _():
                    store(g - 1, b).wait()

                gather(b).start()
            for b in range(_NBUF):
                gather(b).wait()
                store(g, b).start()

                @pl.when(g + 1 < n_groups)
                def _():
                    idx_copy(g + 1, b).start()
            return carry

        lax.fori_loop(0, n_groups, group, 0)
        for b in range(_NBUF):
            store(n_groups - 1, b).wait()

    return gather_kernel


def kernel(index, features):
    n, s = index.shape
    out_t = _make_gather(s, n)(index.T, features)
    return jnp.transpose(out_t, (1, 0, 2))
